# baseline (speedup 1.0000x reference)
import functools

import jax
import jax.numpy as jnp
from jax import lax
from jax.experimental import pallas as pl
from jax.experimental.pallas import tpu as pltpu
from jax.experimental.pallas import tpu_sc as plsc

D = 32
BB = 128
NC = 2
NS = 16
NW = NC * NS


@functools.partial(jax.jit, static_argnums=(2, 3))
def _sc_gather_t(idsT, table, seq, batch):
    mesh = plsc.VectorSubcoreMesh(core_axis_name="c", subcore_axis_name="s")
    ngrp = seq // 2

    @functools.partial(
        pl.kernel,
        out_type=jax.ShapeDtypeStruct((seq, D // 8, batch // BB, 8 * BB),
                                      jnp.float32),
        mesh=mesh,
        scratch_types=(
            [pltpu.VMEM((seq, BB), jnp.int32)]
            + [pltpu.VMEM((2, BB, D), jnp.float32)]
            + [pltpu.VMEM((2, D * BB), jnp.float32)]
            + [pltpu.SemaphoreType.DMA] * 4
        ),
        compiler_params=pltpu.CompilerParams(
            use_tc_tiling_on_sc=False, needs_layout_passes=False
        ),
    )
    def k(ids_hbm, table_hbm, out_hbm, idx_v, buf, bufT, g0, g1, o0, o1):
        gsem, osem = (g0, g1), (o0, o1)
        wid = lax.axis_index("s") * NC + lax.axis_index("c")
        pltpu.sync_copy(
            ids_hbm.at[pl.ds(0, seq), pl.ds(wid * BB, BB)], idx_v
        )

        d_lo = jnp.arange(16, dtype=jnp.int32) * BB + wid * 0
        d_hi = d_lo + 16 * BB

        def start_gather(s, p):
            pltpu.async_copy(table_hbm.at[idx_v.at[s]], buf.at[p], gsem[p])

        def wait_gather(p):
            pltpu.make_async_copy(
                table_hbm.at[pl.ds(0, BB)], buf.at[p], gsem[p]
            ).wait()

        def transpose(p):
            for b in range(BB):
                plsc.store_scatter(
                    bufT.at[p], [d_lo + jnp.int32(b)], buf[p, b, pl.ds(0, 16)]
                )
                plsc.store_scatter(
                    bufT.at[p], [d_hi + jnp.int32(b)], buf[p, b, pl.ds(16, 16)]
                )

        def start_out(s, p):
            for dt in range(D // 8):
                pltpu.async_copy(
                    bufT.at[p, pl.ds(dt * 8 * BB, 8 * BB)],
                    out_hbm.at[s, dt, wid],
                    osem[p],
                )

        def wait_out(p):
            for dt in range(D // 8):
                pltpu.make_async_copy(
                    bufT.at[p, pl.ds(0, 8 * BB)],
                    out_hbm.at[0, 0, 0],
                    osem[p],
                ).wait()

        start_gather(0, 0)
        start_gather(1, 1)

        def body(g, carry):
            for p in (0, 1):
                s = 2 * g + p
                wait_gather(p)

                @pl.when(g > 0)
                def _():
                    wait_out(p)

                transpose(p)
                start_out(s, p)

                @pl.when(g < ngrp - 1)
                def _():
                    start_gather(s + 2, p)

            return carry

        lax.fori_loop(0, ngrp, body, 0)
        wait_out(0)
        wait_out(1)

    return k(idsT, table)


def kernel(relation_ids, table):
    batch, seq = relation_ids.shape
    idsT = relation_ids.T.astype(jnp.int32)
    out4 = _sc_gather_t(idsT, table.astype(jnp.float32), seq, batch)
    out5 = out4.reshape(seq, D // 8, batch // BB, 8, BB)
    return out5.transpose(2, 4, 0, 1, 3).reshape(batch, seq, D)


# speedup vs baseline: 1.3469x; 1.3469x over previous
import functools

import jax
import jax.numpy as jnp
from jax import lax
from jax.experimental import pallas as pl
from jax.experimental.pallas import tpu as pltpu
from jax.experimental.pallas import tpu_sc as plsc

D = 32
BB = 128
NC = 2
NS = 16
NW = NC * NS


@functools.partial(jax.jit, static_argnums=(2, 3))
def _sc_gather_t(idsT, table, seq, batch):
    mesh = plsc.VectorSubcoreMesh(core_axis_name="c", subcore_axis_name="s")
    ngrp = seq // 2

    @functools.partial(
        pl.kernel,
        out_type=jax.ShapeDtypeStruct((seq, D // 8, batch // BB, 8 * BB),
                                      jnp.float32),
        mesh=mesh,
        scratch_types=(
            [pltpu.VMEM((seq, BB), jnp.int32)]
            + [pltpu.VMEM((2, BB, D), jnp.float32)]
            + [pltpu.VMEM((2, D * BB), jnp.float32)]
            + [pltpu.SemaphoreType.DMA] * 4
        ),
        compiler_params=pltpu.CompilerParams(
            use_tc_tiling_on_sc=False, needs_layout_passes=False
        ),
    )
    def k(ids_hbm, table_hbm, out_hbm, idx_v, buf, bufT, g0, g1, o0, o1):
        gsem, osem = (g0, g1), (o0, o1)
        wid = lax.axis_index("s") * NC + lax.axis_index("c")
        pltpu.sync_copy(
            ids_hbm.at[pl.ds(0, seq), pl.ds(wid * BB, BB)], idx_v
        )

        d_lo = jnp.arange(16, dtype=jnp.int32) * BB + wid * 0
        d_hi = d_lo + 16 * BB

        def start_gather(s, p):
            pltpu.async_copy(table_hbm.at[idx_v.at[s]], buf.at[p], gsem[p])

        def wait_gather(p):
            pltpu.make_async_copy(
                table_hbm.at[pl.ds(0, BB)], buf.at[p], gsem[p]
            ).wait()

        def transpose(p):
            @plsc.parallel_loop(0, BB, 1, unroll=8)
            def _(b):
                plsc.store_scatter(
                    bufT.at[p], [d_lo + b], buf[p, b, pl.ds(0, 16)]
                )
                plsc.store_scatter(
                    bufT.at[p], [d_hi + b], buf[p, b, pl.ds(16, 16)]
                )

        def start_out(s, p):
            for dt in range(D // 8):
                pltpu.async_copy(
                    bufT.at[p, pl.ds(dt * 8 * BB, 8 * BB)],
                    out_hbm.at[s, dt, wid],
                    osem[p],
                )

        def wait_out(p):
            for dt in range(D // 8):
                pltpu.make_async_copy(
                    bufT.at[p, pl.ds(0, 8 * BB)],
                    out_hbm.at[0, 0, 0],
                    osem[p],
                ).wait()

        start_gather(0, 0)
        start_gather(1, 1)

        def body(g, carry):
            for p in (0, 1):
                s = 2 * g + p
                wait_gather(p)

                @pl.when(g > 0)
                def _():
                    wait_out(p)

                transpose(p)
                start_out(s, p)

                @pl.when(g < ngrp - 1)
                def _():
                    start_gather(s + 2, p)

            return carry

        lax.fori_loop(0, ngrp, body, 0)
        wait_out(0)
        wait_out(1)

    return k(idsT, table)


def kernel(relation_ids, table):
    batch, seq = relation_ids.shape
    idsT = relation_ids.T.astype(jnp.int32)
    out4 = _sc_gather_t(idsT, table.astype(jnp.float32), seq, batch)
    out5 = out4.reshape(seq, D // 8, batch // BB, 8, BB)
    return out5.transpose(2, 4, 0, 1, 3).reshape(batch, seq, D)


# speedup vs baseline: 2.8252x; 2.0975x over previous
import functools

import jax
import jax.numpy as jnp
from jax import lax
from jax.experimental import pallas as pl
from jax.experimental.pallas import tpu as pltpu
from jax.experimental.pallas import tpu_sc as plsc

D = 32
BB = 128
NC = 2
NS = 16
NW = NC * NS


@functools.partial(jax.jit, static_argnums=(2, 3))
def _sc_gather_t(idsT, table, seq, batch):
    mesh = plsc.VectorSubcoreMesh(core_axis_name="c", subcore_axis_name="s")
    ngrp = seq // 2

    @functools.partial(
        pl.kernel,
        out_type=jax.ShapeDtypeStruct((seq, D // 8, batch // BB, 8 * BB),
                                      jnp.float32),
        mesh=mesh,
        scratch_types=(
            [pltpu.VMEM((seq, BB), jnp.int32)]
            + [pltpu.VMEM((2 * BB, D), jnp.float32)]
            + [pltpu.VMEM((2, D * BB), jnp.float32)]
            + [pltpu.SemaphoreType.DMA] * 4
        ),
        compiler_params=pltpu.CompilerParams(
            use_tc_tiling_on_sc=False, needs_layout_passes=False
        ),
    )
    def k(ids_hbm, table_hbm, out_hbm, idx_v, buf, bufT, g0, g1, o0, o1):
        gsem, osem = (g0, g1), (o0, o1)
        wid = lax.axis_index("s") * NC + lax.axis_index("c")
        pltpu.sync_copy(
            ids_hbm.at[pl.ds(0, seq), pl.ds(wid * BB, BB)], idx_v
        )

        iota16 = jnp.arange(16, dtype=jnp.int32)
        iota_hi = iota16 + 16
        iota_bb = iota16 * BB

        def start_gather(s, p):
            pltpu.async_copy(
                table_hbm.at[idx_v.at[s]],
                buf.at[pl.ds(p * BB, BB)],
                gsem[p],
            )

        def wait_gather(p):
            pltpu.make_async_copy(
                table_hbm.at[pl.ds(0, BB)],
                buf.at[pl.ds(p * BB, BB)],
                gsem[p],
            ).wait()

        def transpose(p):
            @plsc.parallel_loop(0, BB, 1, unroll=8)
            def _(b0):
                wb = lax.bitwise_and(iota16 + b0, BB - 1)
                rows = wb + p * BB
                sidx = iota_bb + wb
                v0 = plsc.load_gather(buf, [rows, iota16])
                plsc.store_scatter(bufT.at[p], [sidx], v0)
                v1 = plsc.load_gather(buf, [rows, iota_hi])
                plsc.store_scatter(bufT.at[p], [sidx + 16 * BB], v1)

        def start_out(s, p):
            for dt in range(D // 8):
                pltpu.async_copy(
                    bufT.at[p, pl.ds(dt * 8 * BB, 8 * BB)],
                    out_hbm.at[s, dt, wid],
                    osem[p],
                )

        def wait_out(p):
            for dt in range(D // 8):
                pltpu.make_async_copy(
                    bufT.at[p, pl.ds(0, 8 * BB)],
                    out_hbm.at[0, 0, 0],
                    osem[p],
                ).wait()

        start_gather(0, 0)
        start_gather(1, 1)

        def body(g, carry):
            for p in (0, 1):
                s = 2 * g + p
                wait_gather(p)

                @pl.when(g > 0)
                def _():
                    wait_out(p)

                transpose(p)
                start_out(s, p)

                @pl.when(g < ngrp - 1)
                def _():
                    start_gather(s + 2, p)

            return carry

        lax.fori_loop(0, ngrp, body, 0)
        wait_out(0)
        wait_out(1)

    return k(idsT, table)


def kernel(relation_ids, table):
    batch, seq = relation_ids.shape
    idsT = relation_ids.T.astype(jnp.int32)
    out4 = _sc_gather_t(idsT, table.astype(jnp.float32), seq, batch)
    out5 = out4.reshape(seq, D // 8, batch // BB, 8, BB)
    return out5.transpose(2, 4, 0, 1, 3).reshape(batch, seq, D)


# speedup vs baseline: 3.7805x; 1.3381x over previous
import functools

import jax
import jax.numpy as jnp
from jax import lax
from jax.experimental import pallas as pl
from jax.experimental.pallas import tpu as pltpu
from jax.experimental.pallas import tpu_sc as plsc

D = 32
BB = 128
NC = 2
NS = 16
NW = NC * NS


@functools.partial(jax.jit, static_argnums=(2, 3))
def _sc_gather_t(idsT, table, seq, batch):
    mesh = plsc.VectorSubcoreMesh(core_axis_name="c", subcore_axis_name="s")
    nbuf = 4
    ngrp = seq // nbuf

    @functools.partial(
        pl.kernel,
        out_type=jax.ShapeDtypeStruct((seq, D // 8, batch // BB, 8 * BB),
                                      jnp.float32),
        mesh=mesh,
        scratch_types=(
            [pltpu.VMEM((seq, BB), jnp.int32)]
            + [pltpu.VMEM((4 * BB, D), jnp.float32)]
            + [pltpu.VMEM((4, D * BB), jnp.float32)]
            + [pltpu.SemaphoreType.DMA] * 8
        ),
        compiler_params=pltpu.CompilerParams(
            use_tc_tiling_on_sc=False, needs_layout_passes=False
        ),
    )
    def k(ids_hbm, table_hbm, out_hbm, idx_v, buf, bufT, *sems):
        gsem, osem = sems[:nbuf], sems[nbuf:]
        wid = lax.axis_index("s") * NC + lax.axis_index("c")
        pltpu.sync_copy(
            ids_hbm.at[pl.ds(0, seq), pl.ds(wid * BB, BB)], idx_v
        )

        iota16 = jnp.arange(16, dtype=jnp.int32)
        iota_hi = iota16 + 16
        iota_bb = iota16 * BB

        def start_gather(s, p):
            pltpu.async_copy(
                table_hbm.at[idx_v.at[s]],
                buf.at[pl.ds(p * BB, BB)],
                gsem[p],
            )

        def wait_gather(p):
            pltpu.make_async_copy(
                table_hbm.at[pl.ds(0, BB)],
                buf.at[pl.ds(p * BB, BB)],
                gsem[p],
            ).wait()

        def transpose(p):
            @plsc.parallel_loop(0, BB, 1, unroll=8)
            def _(b0):
                wb = lax.bitwise_and(iota16 + b0, BB - 1)
                rows = wb + p * BB
                sidx = iota_bb + wb
                v0 = plsc.load_gather(buf, [rows, iota16])
                plsc.store_scatter(bufT.at[p], [sidx], v0)
                v1 = plsc.load_gather(buf, [rows, iota_hi])
                plsc.store_scatter(bufT.at[p], [sidx + 16 * BB], v1)

        def start_out(s, p):
            for dt in range(D // 8):
                pltpu.async_copy(
                    bufT.at[p, pl.ds(dt * 8 * BB, 8 * BB)],
                    out_hbm.at[s, dt, wid],
                    osem[p],
                )

        def wait_out(p):
            for dt in range(D // 8):
                pltpu.make_async_copy(
                    bufT.at[p, pl.ds(0, 8 * BB)],
                    out_hbm.at[0, 0, 0],
                    osem[p],
                ).wait()

        for p in range(nbuf):
            start_gather(p, p)

        def body(g, carry):
            for p in range(nbuf):
                s = nbuf * g + p
                wait_gather(p)

                @pl.when(g > 0)
                def _():
                    wait_out(p)

                transpose(p)
                start_out(s, p)

                @pl.when(g < ngrp - 1)
                def _():
                    start_gather(s + nbuf, p)

            return carry

        lax.fori_loop(0, ngrp, body, 0)
        for p in range(nbuf):
            wait_out(p)

    return k(idsT, table)


def kernel(relation_ids, table):
    batch, seq = relation_ids.shape
    idsT = relation_ids.T.astype(jnp.int32)
    out4 = _sc_gather_t(idsT, table.astype(jnp.float32), seq, batch)
    out5 = out4.reshape(seq, D // 8, batch // BB, 8, BB)
    return out5.transpose(2, 4, 0, 1, 3).reshape(batch, seq, D)


# speedup vs baseline: 3.9893x; 1.0552x over previous
import functools

import jax
import jax.numpy as jnp
from jax import lax
from jax.experimental import pallas as pl
from jax.experimental.pallas import tpu as pltpu
from jax.experimental.pallas import tpu_sc as plsc

D = 32
BB = 128
NC = 2
NS = 16
NW = NC * NS


@functools.partial(jax.jit, static_argnums=(2, 3))
def _sc_gather_t(idsT, table, seq, batch):
    mesh = plsc.VectorSubcoreMesh(core_axis_name="c", subcore_axis_name="s")
    nbuf = 8
    ngrp = seq // nbuf

    @functools.partial(
        pl.kernel,
        out_type=jax.ShapeDtypeStruct((seq, D // 8, batch // BB, 8 * BB),
                                      jnp.float32),
        mesh=mesh,
        scratch_types=(
            [pltpu.VMEM((seq, BB), jnp.int32)]
            + [pltpu.VMEM((8 * BB, D), jnp.float32)]
            + [pltpu.VMEM((8, D * BB), jnp.float32)]
            + [pltpu.SemaphoreType.DMA] * 16
        ),
        compiler_params=pltpu.CompilerParams(
            use_tc_tiling_on_sc=False, needs_layout_passes=False
        ),
    )
    def k(ids_hbm, table_hbm, out_hbm, idx_v, buf, bufT, *sems):
        gsem, osem = sems[:nbuf], sems[nbuf:]
        wid = lax.axis_index("s") * NC + lax.axis_index("c")
        pltpu.sync_copy(
            ids_hbm.at[pl.ds(0, seq), pl.ds(wid * BB, BB)], idx_v
        )

        iota16 = jnp.arange(16, dtype=jnp.int32)
        iota_hi = iota16 + 16
        iota_bb = iota16 * BB

        def start_gather(s, p):
            pltpu.async_copy(
                table_hbm.at[idx_v.at[s]],
                buf.at[pl.ds(p * BB, BB)],
                gsem[p],
            )

        def wait_gather(p):
            pltpu.make_async_copy(
                table_hbm.at[pl.ds(0, BB)],
                buf.at[pl.ds(p * BB, BB)],
                gsem[p],
            ).wait()

        def transpose(p):
            @plsc.parallel_loop(0, BB, 1, unroll=8)
            def _(b0):
                wb = lax.bitwise_and(iota16 + b0, BB - 1)
                rows = wb + p * BB
                sidx = iota_bb + wb
                v0 = plsc.load_gather(buf, [rows, iota16])
                plsc.store_scatter(bufT.at[p], [sidx], v0)
                v1 = plsc.load_gather(buf, [rows, iota_hi])
                plsc.store_scatter(bufT.at[p], [sidx + 16 * BB], v1)

        def start_out(s, p):
            for dt in range(D // 8):
                pltpu.async_copy(
                    bufT.at[p, pl.ds(dt * 8 * BB, 8 * BB)],
                    out_hbm.at[s, dt, wid],
                    osem[p],
                )

        def wait_out(p):
            for dt in range(D // 8):
                pltpu.make_async_copy(
                    bufT.at[p, pl.ds(0, 8 * BB)],
                    out_hbm.at[0, 0, 0],
                    osem[p],
                ).wait()

        for p in range(nbuf):
            start_gather(p, p)

        def body(g, carry):
            for p in range(nbuf):
                s = nbuf * g + p
                wait_gather(p)

                @pl.when(g > 0)
                def _():
                    wait_out(p)

                transpose(p)
                start_out(s, p)

                @pl.when(g < ngrp - 1)
                def _():
                    start_gather(s + nbuf, p)

            return carry

        lax.fori_loop(0, ngrp, body, 0)
        for p in range(nbuf):
            wait_out(p)

    return k(idsT, table)


def kernel(relation_ids, table):
    batch, seq = relation_ids.shape
    idsT = relation_ids.T.astype(jnp.int32)
    out4 = _sc_gather_t(idsT, table.astype(jnp.float32), seq, batch)
    out5 = out4.reshape(seq, D // 8, batch // BB, 8, BB)
    return out5.transpose(2, 4, 0, 1, 3).reshape(batch, seq, D)


# speedup vs baseline: 3.9972x; 1.0020x over previous
import functools

import jax
import jax.numpy as jnp
from jax import lax
from jax.experimental import pallas as pl
from jax.experimental.pallas import tpu as pltpu
from jax.experimental.pallas import tpu_sc as plsc

D = 32
BB = 128
NC = 2
NS = 16
NW = NC * NS


@functools.partial(jax.jit, static_argnums=(2, 3))
def _sc_gather_t(idsT, table, seq, batch):
    mesh = plsc.VectorSubcoreMesh(core_axis_name="c", subcore_axis_name="s")
    nbuf = 10
    ngrp = seq // nbuf

    @functools.partial(
        pl.kernel,
        out_type=jax.ShapeDtypeStruct((seq, D // 8, batch // BB, 8 * BB),
                                      jnp.float32),
        mesh=mesh,
        scratch_types=(
            [pltpu.VMEM((seq, BB), jnp.int32)]
            + [pltpu.VMEM((10 * BB, D), jnp.float32)]
            + [pltpu.VMEM((10, D * BB), jnp.float32)]
            + [pltpu.SemaphoreType.DMA] * 20
        ),
        compiler_params=pltpu.CompilerParams(
            use_tc_tiling_on_sc=False, needs_layout_passes=False
        ),
    )
    def k(ids_hbm, table_hbm, out_hbm, idx_v, buf, bufT, *sems):
        gsem, osem = sems[:nbuf], sems[nbuf:]
        wid = lax.axis_index("s") * NC + lax.axis_index("c")
        pltpu.sync_copy(
            ids_hbm.at[pl.ds(0, seq), pl.ds(wid * BB, BB)], idx_v
        )

        iota16 = jnp.arange(16, dtype=jnp.int32)
        iota_hi = iota16 + 16
        iota_bb = iota16 * BB

        def start_gather(s, p):
            pltpu.async_copy(
                table_hbm.at[idx_v.at[s]],
                buf.at[pl.ds(p * BB, BB)],
                gsem[p],
            )

        def wait_gather(p):
            pltpu.make_async_copy(
                table_hbm.at[pl.ds(0, BB)],
                buf.at[pl.ds(p * BB, BB)],
                gsem[p],
            ).wait()

        def transpose(p):
            @plsc.parallel_loop(0, BB, 1, unroll=8)
            def _(b0):
                wb = lax.bitwise_and(iota16 + b0, BB - 1)
                rows = wb + p * BB
                sidx = iota_bb + wb
                v0 = plsc.load_gather(buf, [rows, iota16])
                plsc.store_scatter(bufT.at[p], [sidx], v0)
                v1 = plsc.load_gather(buf, [rows, iota_hi])
                plsc.store_scatter(bufT.at[p], [sidx + 16 * BB], v1)

        def start_out(s, p):
            for dt in range(D // 8):
                pltpu.async_copy(
                    bufT.at[p, pl.ds(dt * 8 * BB, 8 * BB)],
                    out_hbm.at[s, dt, wid],
                    osem[p],
                )

        def wait_out(p):
            for dt in range(D // 8):
                pltpu.make_async_copy(
                    bufT.at[p, pl.ds(0, 8 * BB)],
                    out_hbm.at[0, 0, 0],
                    osem[p],
                ).wait()

        for p in range(nbuf):
            start_gather(p, p)

        def body(g, carry):
            for p in range(nbuf):
                s = nbuf * g + p
                wait_gather(p)

                @pl.when(g > 0)
                def _():
                    wait_out(p)

                transpose(p)
                start_out(s, p)

                @pl.when(g < ngrp - 1)
                def _():
                    start_gather(s + nbuf, p)

            return carry

        lax.fori_loop(0, ngrp, body, 0)
        for p in range(nbuf):
            wait_out(p)

    return k(idsT, table)


def kernel(relation_ids, table):
    batch, seq = relation_ids.shape
    idsT = relation_ids.T.astype(jnp.int32)
    out4 = _sc_gather_t(idsT, table.astype(jnp.float32), seq, batch)
    out5 = out4.reshape(seq, D // 8, batch // BB, 8, BB)
    return out5.transpose(2, 4, 0, 1, 3).reshape(batch, seq, D)
